# Initial kernel scaffold; baseline (speedup 1.0000x reference)
import jax
import jax.numpy as jnp
from jax.experimental import pallas as pl


def kernel(embedded_features, actor_params, critic_params):
    raise NotImplementedError("write your pallas kernel here")


# speedup vs baseline: 6.7560x; 6.7560x over previous
import jax
import jax.numpy as jnp
from jax.experimental import pallas as pl


def _mlp_pairs_kernel(graph_ref, nodes_ref, wg_ref, wa_ref, wb_ref, b1_ref,
                      w2_ref, b2_ref, w3_ref, b3_ref, w4_ref, b4_ref,
                      wc1_ref, bc1_ref, wc2_ref, bc2_ref,
                      pi_ref, value_ref):
    n = nodes_ref.shape[1]
    g = graph_ref[0]
    nodes = nodes_ref[0]

    f32 = jnp.float32
    gw = jnp.dot(g, wg_ref[...], preferred_element_type=f32)
    a = jnp.dot(nodes, wa_ref[...], preferred_element_type=f32)
    bm = jnp.dot(nodes, wb_ref[...], preferred_element_type=f32)
    base = gw + b1_ref[...]

    h = a[:, None, :] + bm[None, :, :] + base[None, :, :]
    h = jnp.tanh(h).reshape(n * n, -1)
    h = jnp.tanh(jnp.dot(h, w2_ref[...], preferred_element_type=f32) + b2_ref[...])
    h = jnp.tanh(jnp.dot(h, w3_ref[...], preferred_element_type=f32) + b3_ref[...])
    logits = jnp.dot(h, w4_ref[...], preferred_element_type=f32) + b4_ref[...]
    logits = logits[:, 0]

    m = jnp.max(logits)
    e = jnp.exp(logits - m)
    pi_ref[0, 0, :] = e / jnp.sum(e)

    hc = jnp.tanh(jnp.dot(g, wc1_ref[...], preferred_element_type=f32) + bc1_ref[...])
    v = jnp.dot(hc, wc2_ref[...], preferred_element_type=f32) + bc2_ref[...]
    value_ref[0] = v


def kernel(embedded_features, actor_params, critic_params):
    B, n1, _ = embedded_features.shape
    n = n1 - 1
    (w1, b1), (w2, b2), (w3, b3), (w4, b4) = actor_params
    (wc1, bc1), (wc2, bc2) = critic_params
    emb = w1.shape[0] // 3

    graph = embedded_features[:, :1, :emb]
    nodes = embedded_features[:, 1:, :emb]
    wg, wa, wb = w1[:emb], w1[emb:2 * emb], w1[2 * emb:]

    row = lambda x: x.reshape(1, -1)
    const2 = lambda b: (0, 0)

    pi, value = pl.pallas_call(
        _mlp_pairs_kernel,
        grid=(B,),
        in_specs=[
            pl.BlockSpec((1, 1, emb), lambda b: (b, 0, 0)),
            pl.BlockSpec((1, n, emb), lambda b: (b, 0, 0)),
            pl.BlockSpec(wg.shape, const2),
            pl.BlockSpec(wa.shape, const2),
            pl.BlockSpec(wb.shape, const2),
            pl.BlockSpec((1, b1.shape[0]), const2),
            pl.BlockSpec(w2.shape, const2),
            pl.BlockSpec((1, b2.shape[0]), const2),
            pl.BlockSpec(w3.shape, const2),
            pl.BlockSpec((1, b3.shape[0]), const2),
            pl.BlockSpec(w4.shape, const2),
            pl.BlockSpec((1, b4.shape[0]), const2),
            pl.BlockSpec(wc1.shape, const2),
            pl.BlockSpec((1, bc1.shape[0]), const2),
            pl.BlockSpec(wc2.shape, const2),
            pl.BlockSpec((1, bc2.shape[0]), const2),
        ],
        out_specs=[
            pl.BlockSpec((1, 1, n * n), lambda b: (b, 0, 0)),
            pl.BlockSpec((1, 1, 1), lambda b: (b, 0, 0)),
        ],
        out_shape=[
            jax.ShapeDtypeStruct((B, 1, n * n), jnp.float32),
            jax.ShapeDtypeStruct((B, 1, 1), jnp.float32),
        ],
    )(graph, nodes, wg, wa, wb, row(b1), w2, row(b2), w3, row(b3),
      w4, row(b4), wc1, row(bc1), wc2, row(bc2))

    max_n = 100
    shaped = pi.reshape(B, n, n)
    filled = jnp.zeros((B, max_n, max_n), jnp.float32).at[:, :n, :n].set(shaped)
    return (filled.reshape(B, max_n * max_n), value)


# speedup vs baseline: 8.1362x; 1.2043x over previous
import numpy as np
import jax
import jax.numpy as jnp
from jax.experimental import pallas as pl


def _mlp_pairs_kernel(graph_t_ref, nodes_t_ref, cm_ref,
                      wgt_ref, wat_ref, wbt_ref, b1c_ref,
                      w2t_ref, b2c_ref, w3t_ref, b3c_ref, w4t_ref, b4c_ref,
                      wc1t_ref, bc1c_ref, wc2t_ref, bc2c_ref,
                      pi_ref, value_ref):
    f32 = jnp.float32
    g_t = graph_t_ref[0]
    nodes_t = nodes_t_ref[0]

    base = jnp.dot(wgt_ref[...], g_t, preferred_element_type=f32) + b1c_ref[...]
    a_t = jnp.dot(wat_ref[...], nodes_t, preferred_element_type=f32)
    b_t = jnp.dot(wbt_ref[...], nodes_t, preferred_element_type=f32)
    stacked = jnp.concatenate([a_t, b_t, base], axis=1)

    h = jnp.tanh(jnp.dot(stacked, cm_ref[...], preferred_element_type=f32))
    h = jnp.tanh(jnp.dot(w2t_ref[...], h, preferred_element_type=f32) + b2c_ref[...])
    h = jnp.tanh(jnp.dot(w3t_ref[...], h, preferred_element_type=f32) + b3c_ref[...])
    logits = jnp.dot(w4t_ref[...], h, preferred_element_type=f32) + b4c_ref[...]

    e = jnp.exp(logits - jnp.max(logits))
    pi_ref[0] = e / jnp.sum(e)

    hc = jnp.tanh(jnp.dot(wc1t_ref[...], g_t, preferred_element_type=f32) + bc1c_ref[...])
    value_ref[0] = jnp.dot(wc2t_ref[...], hc, preferred_element_type=f32) + bc2c_ref[...]


def kernel(embedded_features, actor_params, critic_params):
    B, n1, _ = embedded_features.shape
    n = n1 - 1
    (w1, b1), (w2, b2), (w3, b3), (w4, b4) = actor_params
    (wc1, bc1), (wc2, bc2) = critic_params
    emb = w1.shape[0] // 3

    graph_t = embedded_features[:, :1, :emb].transpose(0, 2, 1)
    nodes_t = embedded_features[:, 1:, :emb].transpose(0, 2, 1)
    wgt = w1[:emb].T
    wat = w1[emb:2 * emb].T
    wbt = w1[2 * emb:].T

    e_mat = np.repeat(np.eye(n, dtype=np.float32), n, axis=1)
    t_mat = np.tile(np.eye(n, dtype=np.float32), (1, n))
    cm = jnp.asarray(np.concatenate([e_mat, t_mat, np.ones((1, n * n), np.float32)], axis=0))

    col = lambda x: x.reshape(-1, 1)
    const2 = lambda b: (0, 0)

    pi, value = pl.pallas_call(
        _mlp_pairs_kernel,
        grid=(B,),
        in_specs=[
            pl.BlockSpec((1, emb, 1), lambda b: (b, 0, 0)),
            pl.BlockSpec((1, emb, n), lambda b: (b, 0, 0)),
            pl.BlockSpec(cm.shape, const2),
            pl.BlockSpec(wgt.shape, const2),
            pl.BlockSpec(wat.shape, const2),
            pl.BlockSpec(wbt.shape, const2),
            pl.BlockSpec((b1.shape[0], 1), const2),
            pl.BlockSpec(w2.shape, const2),
            pl.BlockSpec((b2.shape[0], 1), const2),
            pl.BlockSpec(w3.shape, const2),
            pl.BlockSpec((b3.shape[0], 1), const2),
            pl.BlockSpec((1, w4.shape[0]), const2),
            pl.BlockSpec((1, 1), const2),
            pl.BlockSpec((wc1.shape[1], wc1.shape[0]), const2),
            pl.BlockSpec((bc1.shape[0], 1), const2),
            pl.BlockSpec((1, wc2.shape[0]), const2),
            pl.BlockSpec((1, 1), const2),
        ],
        out_specs=[
            pl.BlockSpec((1, 1, n * n), lambda b: (b, 0, 0)),
            pl.BlockSpec((1, 1, 1), lambda b: (b, 0, 0)),
        ],
        out_shape=[
            jax.ShapeDtypeStruct((B, 1, n * n), jnp.float32),
            jax.ShapeDtypeStruct((B, 1, 1), jnp.float32),
        ],
    )(graph_t, nodes_t, cm, wgt, wat, wbt, col(b1),
      w2.T, col(b2), w3.T, col(b3), w4.T, b4.reshape(1, 1),
      wc1.T, col(bc1), wc2.T, bc2.reshape(1, 1))

    max_n = 100
    shaped = pi.reshape(B, n, n)
    filled = jnp.zeros((B, max_n, max_n), jnp.float32).at[:, :n, :n].set(shaped)
    return (filled.reshape(B, max_n * max_n), value)


# speedup vs baseline: 10.4961x; 1.2901x over previous
import numpy as np
import jax
import jax.numpy as jnp
from jax.experimental import pallas as pl


def _mlp_pairs_kernel(ef_ref, cm_ref, w1_ref, b1_ref, w2_ref, b2_ref,
                      w3_ref, b3_ref, w4_ref, b4_ref,
                      wc1_ref, bc1_ref, wc2_ref, bc2_ref,
                      pi_ref, value_ref):
    f32 = jnp.float32
    emb = w1_ref.shape[0] // 3
    feats = ef_ref[0]
    f = feats[:, :emb]
    g = f[0:1]

    af = jnp.dot(f, w1_ref[emb:2 * emb], preferred_element_type=f32)
    bf = jnp.dot(f, w1_ref[2 * emb:], preferred_element_type=f32)
    bs = jnp.dot(g, w1_ref[:emb], preferred_element_type=f32) + b1_ref[...]
    stacked = jnp.concatenate([af, bf, bs], axis=0).T

    h = jnp.tanh(jnp.dot(stacked, cm_ref[...], preferred_element_type=f32))
    h = jnp.tanh(jnp.dot(w2_ref[...].T, h, preferred_element_type=f32) + b2_ref[...].T)
    h = jnp.tanh(jnp.dot(w3_ref[...].T, h, preferred_element_type=f32) + b3_ref[...].T)
    logits = jnp.dot(w4_ref[...].T, h, preferred_element_type=f32) + b4_ref[...]

    e = jnp.exp(logits - jnp.max(logits))
    pi_ref[0] = e / jnp.sum(e)

    hc = jnp.tanh(jnp.dot(g, wc1_ref[...], preferred_element_type=f32) + bc1_ref[...])
    value_ref[0] = jnp.dot(hc, wc2_ref[...], preferred_element_type=f32) + bc2_ref[...]


def kernel(embedded_features, actor_params, critic_params):
    B, n1, _ = embedded_features.shape
    n = n1 - 1
    (w1, b1), (w2, b2), (w3, b3), (w4, b4) = actor_params
    (wc1, bc1), (wc2, bc2) = critic_params

    eye = np.eye(n, dtype=np.float32)
    zrow = np.zeros((1, n * n), np.float32)
    ep = np.concatenate([zrow, np.repeat(eye, n, axis=1)], axis=0)
    tp = np.concatenate([zrow, np.tile(eye, (1, n))], axis=0)
    cm = jnp.asarray(np.concatenate([ep, tp, np.ones((1, n * n), np.float32)], axis=0))

    row = lambda x: x.reshape(1, -1)
    const2 = lambda b: (0, 0)
    full = lambda a: pl.BlockSpec(a.shape, const2)

    b1r, b2r, b3r = row(b1), row(b2), row(b3)
    b4r, bc1r, bc2r = b4.reshape(1, 1), row(bc1), bc2.reshape(1, 1)

    pi, value = pl.pallas_call(
        _mlp_pairs_kernel,
        grid=(B,),
        in_specs=[
            pl.BlockSpec((1,) + embedded_features.shape[1:], lambda b: (b, 0, 0)),
            full(cm), full(w1), full(b1r), full(w2), full(b2r),
            full(w3), full(b3r), full(w4), full(b4r),
            full(wc1), full(bc1r), full(wc2), full(bc2r),
        ],
        out_specs=[
            pl.BlockSpec((1, 1, n * n), lambda b: (b, 0, 0)),
            pl.BlockSpec((1, 1, 1), lambda b: (b, 0, 0)),
        ],
        out_shape=[
            jax.ShapeDtypeStruct((B, 1, n * n), jnp.float32),
            jax.ShapeDtypeStruct((B, 1, 1), jnp.float32),
        ],
    )(embedded_features, cm, w1, b1r, w2, b2r, w3, b3r, w4, b4r,
      wc1, bc1r, wc2, bc2r)

    return (pi.reshape(B, n * n), value)
